# Initial kernel scaffold; baseline (speedup 1.0000x reference)
import jax
import jax.numpy as jnp
from jax.experimental import pallas as pl


def kernel(pair, embedding_table, bias_table):
    raise NotImplementedError("write your pallas kernel here")


# speedup vs baseline: 1.3780x; 1.3780x over previous
import functools

import jax
import jax.numpy as jnp
from jax import lax
from jax.experimental import pallas as pl
from jax.experimental.pallas import tpu as pltpu
from jax.experimental.pallas import tpu_sc as plsc

NB_EMBEDDINGS = 100000
EMB_DIM = 64
BATCH = 16384

NUM_CORES = 2
NUM_SUBCORES = 16
LANES = 16
NUM_WORKERS = NUM_CORES * NUM_SUBCORES
BPW = BATCH // NUM_WORKERS
CHUNK = 128
NCHUNKS = BPW // CHUNK
GROUPS = BPW // LANES


def _sc_body(idx0_hbm, idx1_hbm, table_hbm, bias_hbm, out_hbm,
             idx0_v, idx1_v, rows0_v, rows1_v, b0_v, b1_v, out_v, sem):
  wid = lax.axis_index("s") * NUM_CORES + lax.axis_index("c")
  base = wid * BPW

  pltpu.sync_copy(idx0_hbm.at[pl.ds(base, BPW)], idx0_v)
  pltpu.sync_copy(idx1_hbm.at[pl.ds(base, BPW)], idx1_v)

  copies = []
  for c in range(NCHUNKS):
    sl = pl.ds(c * CHUNK, CHUNK)
    copies.append(pltpu.async_copy(
        table_hbm.at[idx0_v.at[sl]], rows0_v.at[sl], sem))
    copies.append(pltpu.async_copy(
        table_hbm.at[idx1_v.at[sl]], rows1_v.at[sl], sem))
    copies.append(pltpu.async_copy(
        bias_hbm.at[idx0_v.at[sl]], b0_v.at[sl], sem))
    copies.append(pltpu.async_copy(
        bias_hbm.at[idx1_v.at[sl]], b1_v.at[sl], sem))
  for cp in copies:
    cp.wait()

  lane = lax.iota(jnp.int32, 16)

  def group_body(g, _):
    row_ids = g * LANES + lane

    def dim_body(d, acc):
      col = jnp.zeros((16,), jnp.int32) + d
      a = plsc.load_gather(rows0_v, [row_ids, col])
      b = plsc.load_gather(rows1_v, [row_ids, col])
      return acc + a * b

    acc = lax.fori_loop(0, EMB_DIM, dim_body, jnp.zeros((16,), jnp.float32))
    res = acc + b0_v[pl.ds(g * LANES, LANES)] + b1_v[pl.ds(g * LANES, LANES)]
    out_v[pl.ds(g * LANES, LANES)] = res
    return 0

  lax.fori_loop(0, GROUPS, group_body, 0)

  pltpu.sync_copy(out_v, out_hbm.at[pl.ds(base, BPW)])


@jax.jit
def _run(idx0, idx1, table, bias):
  mesh = plsc.VectorSubcoreMesh(core_axis_name="c", subcore_axis_name="s")
  f = pl.kernel(
      _sc_body,
      out_type=jax.ShapeDtypeStruct((BATCH,), jnp.float32),
      mesh=mesh,
      scratch_types=[
          pltpu.VMEM((BPW,), jnp.int32),
          pltpu.VMEM((BPW,), jnp.int32),
          pltpu.VMEM((BPW, EMB_DIM), jnp.float32),
          pltpu.VMEM((BPW, EMB_DIM), jnp.float32),
          pltpu.VMEM((BPW,), jnp.float32),
          pltpu.VMEM((BPW,), jnp.float32),
          pltpu.VMEM((BPW,), jnp.float32),
          pltpu.SemaphoreType.DMA,
      ],
      compiler_params=pltpu.CompilerParams(
          needs_layout_passes=False, use_tc_tiling_on_sc=False),
  )
  return f(idx0, idx1, table, bias)


def kernel(pair, embedding_table, bias_table):
  idx0 = pair[:, 0].astype(jnp.int32)
  idx1 = (pair[:, 1] + NB_EMBEDDINGS).astype(jnp.int32)
  bias_flat = bias_table.reshape(-1)
  sim = _run(idx0, idx1, embedding_table, bias_flat)
  return sim.reshape(BATCH, 1)


# speedup vs baseline: 1.3829x; 1.0035x over previous
import functools

import jax
import jax.numpy as jnp
from jax import lax
from jax.experimental import pallas as pl
from jax.experimental.pallas import tpu as pltpu
from jax.experimental.pallas import tpu_sc as plsc

NB_EMBEDDINGS = 100000
EMB_DIM = 64
BATCH = 16384

NUM_CORES = 2
NUM_SUBCORES = 16
LANES = 16
NUM_WORKERS = NUM_CORES * NUM_SUBCORES
BPW = BATCH // NUM_WORKERS
CHUNK = 128
NCHUNKS = BPW // CHUNK
GROUPS = BPW // LANES


def _sc_body(idx0_hbm, idx1_hbm, table_hbm, bias_hbm, out_hbm,
             idx0_v, idx1_v, rows0_v, rows1_v, b0_v, b1_v, out_v, sem):
  wid = lax.axis_index("s") * NUM_CORES + lax.axis_index("c")
  base = wid * BPW

  pltpu.sync_copy(idx0_hbm.at[pl.ds(base, BPW)], idx0_v)
  pltpu.sync_copy(idx1_hbm.at[pl.ds(base, BPW)], idx1_v)

  copies = []
  for c in range(NCHUNKS):
    sl = pl.ds(c * CHUNK, CHUNK)
    copies.append(pltpu.async_copy(
        table_hbm.at[idx0_v.at[sl]], rows0_v.at[sl], sem))
    copies.append(pltpu.async_copy(
        table_hbm.at[idx1_v.at[sl]], rows1_v.at[sl], sem))
    copies.append(pltpu.async_copy(
        bias_hbm.at[idx0_v.at[sl]], b0_v.at[sl], sem))
    copies.append(pltpu.async_copy(
        bias_hbm.at[idx1_v.at[sl]], b1_v.at[sl], sem))
  for cp in copies:
    cp.wait()

  lane = lax.iota(jnp.int32, 16)

  def group_body(g, _):
    row_ids = g * LANES + lane

    acc = jnp.zeros((16,), jnp.float32)
    for d in range(EMB_DIM):
      col = jnp.full((16,), d, jnp.int32)
      a = plsc.load_gather(rows0_v, [row_ids, col])
      b = plsc.load_gather(rows1_v, [row_ids, col])
      acc = acc + a * b
    res = acc + b0_v[pl.ds(g * LANES, LANES)] + b1_v[pl.ds(g * LANES, LANES)]
    out_v[pl.ds(g * LANES, LANES)] = res
    return 0

  lax.fori_loop(0, GROUPS, group_body, 0)

  pltpu.sync_copy(out_v, out_hbm.at[pl.ds(base, BPW)])


@jax.jit
def _run(idx0, idx1, table, bias):
  mesh = plsc.VectorSubcoreMesh(core_axis_name="c", subcore_axis_name="s")
  f = pl.kernel(
      _sc_body,
      out_type=jax.ShapeDtypeStruct((BATCH,), jnp.float32),
      mesh=mesh,
      scratch_types=[
          pltpu.VMEM((BPW,), jnp.int32),
          pltpu.VMEM((BPW,), jnp.int32),
          pltpu.VMEM((BPW, EMB_DIM), jnp.float32),
          pltpu.VMEM((BPW, EMB_DIM), jnp.float32),
          pltpu.VMEM((BPW,), jnp.float32),
          pltpu.VMEM((BPW,), jnp.float32),
          pltpu.VMEM((BPW,), jnp.float32),
          pltpu.SemaphoreType.DMA,
      ],
      compiler_params=pltpu.CompilerParams(
          needs_layout_passes=False, use_tc_tiling_on_sc=False),
  )
  return f(idx0, idx1, table, bias)


def kernel(pair, embedding_table, bias_table):
  idx0 = pair[:, 0].astype(jnp.int32)
  idx1 = (pair[:, 1] + NB_EMBEDDINGS).astype(jnp.int32)
  bias_flat = bias_table.reshape(-1)
  sim = _run(idx0, idx1, embedding_table, bias_flat)
  return sim.reshape(BATCH, 1)
